# Initial kernel scaffold; baseline (speedup 1.0000x reference)
import jax
import jax.numpy as jnp
from jax.experimental import pallas as pl


def kernel(x, table):
    raise NotImplementedError("write your pallas kernel here")


# speedup vs baseline: 1.2123x; 1.2123x over previous
import functools

import jax
import jax.numpy as jnp
from jax import lax
from jax.experimental import pallas as pl
from jax.experimental.pallas import tpu as pltpu
from jax.experimental.pallas import tpu_sc as plsc

VOCAB = 100000
EMB = 64
BATCH = 4096
FIELDS = 26
BFLAT = BATCH * FIELDS

NC = 2
NS = 16
NW = NC * NS
B_PER_W = BFLAT // NW
CHUNK = 832
NCHUNK = B_PER_W // CHUNK

_mesh = plsc.VectorSubcoreMesh(core_axis_name="c", subcore_axis_name="s")


@functools.partial(
    pl.kernel,
    mesh=_mesh,
    out_type=jax.ShapeDtypeStruct((BFLAT, EMB), jnp.float32),
    compiler_params=pltpu.CompilerParams(use_tc_tiling_on_sc=False),
    scratch_types=[
        pltpu.VMEM((B_PER_W,), jnp.int32),
        pltpu.VMEM((CHUNK, EMB), jnp.float32),
        pltpu.SemaphoreType.DMA,
    ],
)
def _emb_gather(idx_hbm, table_hbm, out_hbm, idx_v, rows_v, sem):
    wid = lax.axis_index("s") * NC + lax.axis_index("c")
    base = wid * B_PER_W
    pltpu.sync_copy(idx_hbm.at[pl.ds(base, B_PER_W)], idx_v)
    for ci in range(NCHUNK):
        pltpu.async_copy(
            table_hbm.at[idx_v.at[pl.ds(ci * CHUNK, CHUNK)]], rows_v, sem
        ).wait()
        pltpu.sync_copy(rows_v, out_hbm.at[pl.ds(base + ci * CHUNK, CHUNK)])


def kernel(x, table):
    idx = x.reshape(BFLAT).astype(jnp.int32)
    out = _emb_gather(idx, table)
    return out.reshape(BATCH, FIELDS, EMB)


# speedup vs baseline: 1.2202x; 1.0065x over previous
import functools

import jax
import jax.numpy as jnp
from jax import lax
from jax.experimental import pallas as pl
from jax.experimental.pallas import tpu as pltpu
from jax.experimental.pallas import tpu_sc as plsc

VOCAB = 100000
EMB = 64
BATCH = 4096
FIELDS = 26
BFLAT = BATCH * FIELDS

NC = 2
NS = 16
NW = NC * NS
B_PER_W = BFLAT // NW
CHUNK = 416
NCHUNK = B_PER_W // CHUNK
NBUF = 4

_mesh = plsc.VectorSubcoreMesh(core_axis_name="c", subcore_axis_name="s")


@functools.partial(
    pl.kernel,
    mesh=_mesh,
    out_type=jax.ShapeDtypeStruct((BFLAT, EMB), jnp.float32),
    compiler_params=pltpu.CompilerParams(use_tc_tiling_on_sc=False),
    scratch_types=[
        pltpu.VMEM((B_PER_W,), jnp.int32),
        pltpu.VMEM((NBUF, CHUNK, EMB), jnp.float32),
        pltpu.SemaphoreType.DMA((NBUF,)),
        pltpu.SemaphoreType.DMA((NBUF,)),
    ],
)
def _emb_gather(idx_hbm, table_hbm, out_hbm, idx_v, rows_v, gsem, ssem):
    wid = lax.axis_index("s") * NC + lax.axis_index("c")
    base = wid * B_PER_W
    pltpu.sync_copy(idx_hbm.at[pl.ds(base, B_PER_W)], idx_v)

    def gather(ci, b):
        return pltpu.async_copy(
            table_hbm.at[idx_v.at[pl.ds(ci * CHUNK, CHUNK)]],
            rows_v.at[b],
            gsem.at[b],
        )

    g = {}
    s = {}
    for ci in range(NBUF):
        g[ci] = gather(ci, ci)
    for ci in range(NCHUNK):
        b = ci % NBUF
        g[ci].wait()
        s[ci] = pltpu.async_copy(
            rows_v.at[b], out_hbm.at[pl.ds(base + ci * CHUNK, CHUNK)], ssem.at[b]
        )
        nx = ci + NBUF
        if nx < NCHUNK:
            s[ci].wait()
            g[nx] = gather(nx, b)
    for ci in range(max(0, NCHUNK - NBUF), NCHUNK):
        s[ci].wait()


def kernel(x, table):
    idx = x.reshape(BFLAT).astype(jnp.int32)
    out = _emb_gather(idx, table)
    return out.reshape(BATCH, FIELDS, EMB)
